# Initial kernel scaffold; baseline (speedup 1.0000x reference)
import jax
import jax.numpy as jnp
from jax.experimental import pallas as pl


def kernel(x, edge_index, W_l, b_l, W_r):
    raise NotImplementedError("write your pallas kernel here")


# speedup vs baseline: 7.5337x; 7.5337x over previous
import functools

import jax
import jax.numpy as jnp
from jax import lax
from jax.experimental import pallas as pl
from jax.experimental.pallas import tpu as pltpu
from jax.experimental.pallas import tpu_sc as plsc

N = 10000
E = 320000
D = 128
DP = 144
NC = 2
NS = 16
NW = NC * NS
K = 80
EK = E // K
CH = EK // NW
NP = 10112
NR = NP // NS


def _sc_accumulate():
    mesh = plsc.VectorSubcoreMesh(core_axis_name="c", subcore_axis_name="s")

    @functools.partial(
        pl.kernel,
        out_type=jax.ShapeDtypeStruct((NC, NP, DP), jnp.float32),
        mesh=mesh,
        scratch_types=[
            pltpu.VMEM((CH, K), jnp.int32),
            pltpu.VMEM((CH, K), jnp.int32),
            pltpu.VMEM((K, DP), jnp.float32),
            pltpu.SemaphoreType.DMA,
            pltpu.VMEM_SHARED((NP, DP), jnp.float32),
        ],
        compiler_params=pltpu.CompilerParams(use_tc_tiling_on_sc=False),
    )
    def sc_fn(xaug_hbm, src_hbm, dst_hbm, zrow_hbm, out_hbm,
              src_v, dst_v, rows_v, sem, acc):
        c = lax.axis_index("c")
        s = lax.axis_index("s")
        wid = s * NC + c

        pltpu.sync_copy(zrow_hbm, acc.at[pl.ds(s * NR, NR)])
        pltpu.sync_copy(src_hbm.at[wid], src_v)
        pltpu.sync_copy(dst_hbm.at[wid], dst_v)
        plsc.subcore_barrier()

        def body(j, carry):
            pltpu.async_copy(xaug_hbm.at[src_v.at[j]], rows_v, sem).wait()
            pltpu.sync_copy(rows_v, acc.at[dst_v.at[j]], add=True)
            return carry

        lax.fori_loop(0, CH, body, 0)
        plsc.subcore_barrier()

        pltpu.sync_copy(acc.at[pl.ds(s * NR, NR)],
                        out_hbm.at[c, pl.ds(s * NR, NR)])

    return sc_fn


def _tc_finish(partial, x, W_l, b_l, W_r):
    BN = 1000

    def body(p_ref, x_ref, wl_ref, bl_ref, wr_ref, o_ref):
        p0 = p_ref[0]
        p1 = p_ref[1]
        summed = p0[:, :D] + p1[:, :D]
        cnt = p0[:, D:D + 1] + p1[:, D:D + 1]
        mean = summed / jnp.maximum(cnt, 1.0)
        o_ref[...] = (
            jnp.dot(mean, wl_ref[...], preferred_element_type=jnp.float32)
            + jnp.dot(x_ref[...], wr_ref[...], preferred_element_type=jnp.float32)
            + bl_ref[...]
        )

    return pl.pallas_call(
        body,
        grid=(N // BN,),
        in_specs=[
            pl.BlockSpec((NC, BN, DP), lambda i: (0, i, 0)),
            pl.BlockSpec((BN, D), lambda i: (i, 0)),
            pl.BlockSpec((D, D), lambda i: (0, 0)),
            pl.BlockSpec((1, D), lambda i: (0, 0)),
            pl.BlockSpec((D, D), lambda i: (0, 0)),
        ],
        out_specs=pl.BlockSpec((BN, D), lambda i: (i, 0)),
        out_shape=jax.ShapeDtypeStruct((N, D), jnp.float32),
    )(partial, x, W_l, b_l.reshape(1, D), W_r)


def kernel(x, edge_index, W_l, b_l, W_r):
    xaug = jnp.concatenate(
        [x, jnp.ones((N, 1), jnp.float32), jnp.zeros((N, DP - D - 1), jnp.float32)],
        axis=1,
    )
    src_r = edge_index[0].reshape(NW, CH, K)
    dst_r = edge_index[1].reshape(NW, CH, K)
    zrow = jnp.zeros((NR, DP), jnp.float32)
    partial = _sc_accumulate()(xaug, src_r, dst_r, zrow)
    return _tc_finish(partial, x, W_l, b_l, W_r)


# speedup vs baseline: 12.6068x; 1.6734x over previous
import functools

import jax
import jax.numpy as jnp
from jax import lax
from jax.experimental import pallas as pl
from jax.experimental.pallas import tpu as pltpu
from jax.experimental.pallas import tpu_sc as plsc

N = 10000
E = 320000
D = 128
CW = 8
NC = 2
NS = 16
NW = NC * NS
K = 80
EK = E // K
CH = EK // NW
NP = 10112
NR = NP // NS


def _sc_accumulate():
    mesh = plsc.VectorSubcoreMesh(core_axis_name="c", subcore_axis_name="s")

    @functools.partial(
        pl.kernel,
        out_type=(
            jax.ShapeDtypeStruct((NC, NP, D), jnp.float32),
            jax.ShapeDtypeStruct((NC, NP, CW), jnp.float32),
        ),
        mesh=mesh,
        scratch_types=[
            pltpu.VMEM((CH, K), jnp.int32),
            pltpu.VMEM((CH, K), jnp.int32),
            pltpu.VMEM((K, D), jnp.float32),
            pltpu.VMEM((K, D), jnp.float32),
            pltpu.VMEM((K, CW), jnp.float32),
            pltpu.SemaphoreType.DMA,
            pltpu.SemaphoreType.DMA,
            pltpu.VMEM_SHARED((NP, D), jnp.float32),
            pltpu.VMEM_SHARED((NP, CW), jnp.float32),
        ],
        compiler_params=pltpu.CompilerParams(use_tc_tiling_on_sc=False),
    )
    def sc_fn(x_hbm, src_hbm, dst_hbm, zsum_hbm, zcnt_hbm, ones_hbm,
              osum_hbm, ocnt_hbm,
              src_v, dst_v, rows0, rows1, ones_v, sem0, sem1, acc, cnt):
        c = lax.axis_index("c")
        s = lax.axis_index("s")
        wid = s * NC + c

        pltpu.sync_copy(zsum_hbm, acc.at[pl.ds(s * NR, NR)])
        pltpu.sync_copy(zcnt_hbm, cnt.at[pl.ds(s * NR, NR)])
        pltpu.sync_copy(ones_hbm, ones_v)
        pltpu.sync_copy(src_hbm.at[wid], src_v)
        pltpu.sync_copy(dst_hbm.at[wid], dst_v)
        plsc.subcore_barrier()

        pltpu.async_copy(x_hbm.at[src_v.at[0]], rows0, sem0)

        def body(k, carry):
            j = 2 * k
            pltpu.make_async_copy(x_hbm.at[src_v.at[j]], rows0, sem0).wait()
            pltpu.async_copy(x_hbm.at[src_v.at[j + 1]], rows1, sem1)
            pltpu.sync_copy(rows0, acc.at[dst_v.at[j]], add=True)
            pltpu.sync_copy(ones_v, cnt.at[dst_v.at[j]], add=True)
            pltpu.async_copy(x_hbm.at[src_v.at[j + 2]], rows0, sem0)
            pltpu.make_async_copy(x_hbm.at[src_v.at[j + 1]], rows1, sem1).wait()
            pltpu.sync_copy(rows1, acc.at[dst_v.at[j + 1]], add=True)
            pltpu.sync_copy(ones_v, cnt.at[dst_v.at[j + 1]], add=True)
            return carry

        lax.fori_loop(0, (CH - 1) // 2, body, 0)
        pltpu.make_async_copy(x_hbm.at[src_v.at[CH - 1]], rows0, sem0).wait()
        pltpu.sync_copy(rows0, acc.at[dst_v.at[CH - 1]], add=True)
        pltpu.sync_copy(ones_v, cnt.at[dst_v.at[CH - 1]], add=True)
        plsc.subcore_barrier()

        pltpu.sync_copy(acc.at[pl.ds(s * NR, NR)],
                        osum_hbm.at[c, pl.ds(s * NR, NR)])
        pltpu.sync_copy(cnt.at[pl.ds(s * NR, NR)],
                        ocnt_hbm.at[c, pl.ds(s * NR, NR)])

    return sc_fn


def _tc_finish(psum, pcnt, x, W_l, b_l, W_r):
    BN = 1000

    def body(p_ref, c_ref, x_ref, wl_ref, bl_ref, wr_ref, o_ref):
        summed = p_ref[0] + p_ref[1]
        cnt = c_ref[0][:, 0:1] + c_ref[1][:, 0:1]
        mean = summed / jnp.maximum(cnt, 1.0)
        o_ref[...] = (
            jnp.dot(mean, wl_ref[...], preferred_element_type=jnp.float32)
            + jnp.dot(x_ref[...], wr_ref[...], preferred_element_type=jnp.float32)
            + bl_ref[...]
        )

    return pl.pallas_call(
        body,
        grid=(N // BN,),
        in_specs=[
            pl.BlockSpec((NC, BN, D), lambda i: (0, i, 0)),
            pl.BlockSpec((NC, BN, CW), lambda i: (0, i, 0)),
            pl.BlockSpec((BN, D), lambda i: (i, 0)),
            pl.BlockSpec((D, D), lambda i: (0, 0)),
            pl.BlockSpec((1, D), lambda i: (0, 0)),
            pl.BlockSpec((D, D), lambda i: (0, 0)),
        ],
        out_specs=pl.BlockSpec((BN, D), lambda i: (i, 0)),
        out_shape=jax.ShapeDtypeStruct((N, D), jnp.float32),
    )(psum, pcnt, x, W_l, b_l.reshape(1, D), W_r)


def kernel(x, edge_index, W_l, b_l, W_r):
    src_r = edge_index[0].reshape(NW, CH, K)
    dst_r = edge_index[1].reshape(NW, CH, K)
    zsum = jnp.zeros((NR, D), jnp.float32)
    zcnt = jnp.zeros((NR, CW), jnp.float32)
    ones = jnp.ones((K, CW), jnp.float32)
    psum, pcnt = _sc_accumulate()(x, src_r, dst_r, zsum, zcnt, ones)
    return _tc_finish(psum, pcnt, x, W_l, b_l, W_r)


# speedup vs baseline: 13.3292x; 1.0573x over previous
import functools

import jax
import jax.numpy as jnp
from jax import lax
from jax.experimental import pallas as pl
from jax.experimental.pallas import tpu as pltpu
from jax.experimental.pallas import tpu_sc as plsc

N = 10000
E = 320000
D = 128
CW = 8
NC = 2
NS = 16
NW = NC * NS
K = 80
EK = E // K
CH = EK // NW
NP = 10112
NR = NP // NS


def _sc_accumulate():
    mesh = plsc.VectorSubcoreMesh(core_axis_name="c", subcore_axis_name="s")

    @functools.partial(
        pl.kernel,
        out_type=(
            jax.ShapeDtypeStruct((NC, NP, D), jnp.float32),
            jax.ShapeDtypeStruct((NC, NP, CW), jnp.float32),
        ),
        mesh=mesh,
        scratch_types=[
            pltpu.VMEM((CH, K), jnp.int32),
            pltpu.VMEM((CH, K), jnp.int32),
            pltpu.VMEM((K, D), jnp.float32),
            pltpu.VMEM((K, D), jnp.float32),
            pltpu.VMEM((K, CW), jnp.float32),
            pltpu.SemaphoreType.DMA,
            pltpu.SemaphoreType.DMA,
            pltpu.VMEM_SHARED((NP, D), jnp.float32),
            pltpu.VMEM_SHARED((NP, CW), jnp.float32),
        ],
        compiler_params=pltpu.CompilerParams(use_tc_tiling_on_sc=False),
    )
    def sc_fn(x_hbm, edges_hbm, zsum_hbm, zcnt_hbm, ones_hbm,
              osum_hbm, ocnt_hbm,
              src_v, dst_v, rows0, rows1, ones_v, sem0, sem1, acc, cnt):
        c = lax.axis_index("c")
        s = lax.axis_index("s")
        wid = s * NC + c

        pltpu.sync_copy(zsum_hbm, acc.at[pl.ds(s * NR, NR)])
        pltpu.sync_copy(zcnt_hbm, cnt.at[pl.ds(s * NR, NR)])
        pltpu.sync_copy(ones_hbm, ones_v)
        pltpu.sync_copy(edges_hbm.at[0, wid], src_v)
        pltpu.sync_copy(edges_hbm.at[1, wid], dst_v)
        plsc.subcore_barrier()

        pltpu.async_copy(x_hbm.at[src_v.at[0]], rows0, sem0)

        def body(k, carry):
            j = 2 * k
            pltpu.make_async_copy(x_hbm.at[src_v.at[j]], rows0, sem0).wait()
            pltpu.async_copy(x_hbm.at[src_v.at[j + 1]], rows1, sem1)
            pltpu.sync_copy(rows0, acc.at[dst_v.at[j]], add=True)
            pltpu.sync_copy(ones_v, cnt.at[dst_v.at[j]], add=True)
            pltpu.async_copy(x_hbm.at[src_v.at[j + 2]], rows0, sem0)
            pltpu.make_async_copy(x_hbm.at[src_v.at[j + 1]], rows1, sem1).wait()
            pltpu.sync_copy(rows1, acc.at[dst_v.at[j + 1]], add=True)
            pltpu.sync_copy(ones_v, cnt.at[dst_v.at[j + 1]], add=True)
            return carry

        lax.fori_loop(0, (CH - 1) // 2, body, 0)
        pltpu.make_async_copy(x_hbm.at[src_v.at[CH - 1]], rows0, sem0).wait()
        pltpu.sync_copy(rows0, acc.at[dst_v.at[CH - 1]], add=True)
        pltpu.sync_copy(ones_v, cnt.at[dst_v.at[CH - 1]], add=True)
        plsc.subcore_barrier()

        pltpu.sync_copy(acc.at[pl.ds(s * NR, NR)],
                        osum_hbm.at[c, pl.ds(s * NR, NR)])
        pltpu.sync_copy(cnt.at[pl.ds(s * NR, NR)],
                        ocnt_hbm.at[c, pl.ds(s * NR, NR)])

    return sc_fn


def _tc_self(x, W_r, b_l):
    BN = 2000

    def body(x_ref, wr_ref, bl_ref, o_ref):
        o_ref[...] = (
            jnp.dot(x_ref[...], wr_ref[...], preferred_element_type=jnp.float32)
            + bl_ref[...]
        )

    return pl.pallas_call(
        body,
        grid=(N // BN,),
        in_specs=[
            pl.BlockSpec((BN, D), lambda i: (i, 0)),
            pl.BlockSpec((D, D), lambda i: (0, 0)),
            pl.BlockSpec((1, D), lambda i: (0, 0)),
        ],
        out_specs=pl.BlockSpec((BN, D), lambda i: (i, 0)),
        out_shape=jax.ShapeDtypeStruct((N, D), jnp.float32),
    )(x, W_r, b_l.reshape(1, D))


def _tc_finish(psum, pcnt, selfterm, W_l):
    BN = 2000

    def body(p_ref, c_ref, s_ref, wl_ref, o_ref):
        summed = p_ref[0] + p_ref[1]
        cnt = c_ref[0][:, 0:1] + c_ref[1][:, 0:1]
        mean = summed / jnp.maximum(cnt, 1.0)
        o_ref[...] = (
            jnp.dot(mean, wl_ref[...], preferred_element_type=jnp.float32)
            + s_ref[...]
        )

    return pl.pallas_call(
        body,
        grid=(N // BN,),
        in_specs=[
            pl.BlockSpec((NC, BN, D), lambda i: (0, i, 0)),
            pl.BlockSpec((NC, BN, CW), lambda i: (0, i, 0)),
            pl.BlockSpec((BN, D), lambda i: (i, 0)),
            pl.BlockSpec((D, D), lambda i: (0, 0)),
        ],
        out_specs=pl.BlockSpec((BN, D), lambda i: (i, 0)),
        out_shape=jax.ShapeDtypeStruct((N, D), jnp.float32),
    )(psum, pcnt, selfterm, W_l)


def kernel(x, edge_index, W_l, b_l, W_r):
    edges = edge_index.reshape(2, NW, CH, K)
    zsum = jnp.zeros((NR, D), jnp.float32)
    zcnt = jnp.zeros((NR, CW), jnp.float32)
    ones = jnp.ones((K, CW), jnp.float32)
    selfterm = _tc_self(x, W_r, b_l)
    psum, pcnt = _sc_accumulate()(x, edges, zsum, zcnt, ones)
    return _tc_finish(psum, pcnt, selfterm, W_l)
